# baseline (speedup 1.0000x reference)
import functools

import jax
import jax.numpy as jnp
from jax import lax
from jax.experimental import pallas as pl
from jax.experimental.pallas import tpu as pltpu
from jax.experimental.pallas import tpu_sc as plsc

EMB = 64
CHUNK = 400
SUBS = ((0, 128), (128, 128), (256, 128), (384, 16))


@functools.partial(jax.jit, static_argnames=("total",))
def _sc_embedding_gather(table_pad, idx, total):
    info = plsc.get_sparse_core_info()
    num_workers = info.num_cores * info.num_subcores
    per_worker = total // num_workers
    n_chunks = per_worker // CHUNK
    half_t = n_chunks // 2
    mesh = plsc.VectorSubcoreMesh(core_axis_name="c", subcore_axis_name="s")

    @functools.partial(
        pl.kernel,
        mesh=mesh,
        out_type=jax.ShapeDtypeStruct((total, 2 * EMB), jnp.float32),
        scratch_types=[
            pltpu.VMEM((CHUNK,), jnp.int32),
            pltpu.VMEM((CHUNK,), jnp.int32),
            pltpu.VMEM((CHUNK, 2 * EMB), jnp.float32),
            pltpu.VMEM((CHUNK, 2 * EMB), jnp.float32),
            pltpu.SemaphoreType.DMA,
            pltpu.SemaphoreType.DMA,
            pltpu.SemaphoreType.DMA,
            pltpu.SemaphoreType.DMA,
        ],
    )
    def k(pt_hbm, idx_hbm, out_hbm, idx0, idx1, rows0, rows1,
          gsem0, gsem1, osem0, osem1):
        wid = lax.axis_index("s") * info.num_cores + lax.axis_index("c")
        base = wid * per_worker

        def fire_gather(idx_v, rows_v, gsem):
            for off, sz in SUBS:
                pltpu.async_copy(
                    pt_hbm.at[idx_v.at[pl.ds(off, sz)]],
                    rows_v.at[pl.ds(off, sz)],
                    gsem,
                )

        def drain_gather(rows_v, gsem):
            pltpu.make_async_copy(out_hbm.at[pl.ds(0, CHUNK)], rows_v, gsem).wait()

        def drain_out(rows_v, osem):
            pltpu.make_async_copy(rows_v, out_hbm.at[pl.ds(0, CHUNK)], osem).wait()

        def load_idx(g, idx_v):
            pltpu.sync_copy(idx_hbm.at[pl.ds(base + g * CHUNK, CHUNK)], idx_v)

        def fire_out(g, rows_v, osem):
            pltpu.async_copy(rows_v, out_hbm.at[pl.ds(base + g * CHUNK, CHUNK)], osem)

        load_idx(0, idx0)
        fire_gather(idx0, rows0, gsem0)

        def body(t, carry):
            g = 2 * t

            @pl.when(t > 0)
            def _():
                drain_out(rows1, osem1)

            load_idx(g + 1, idx1)
            fire_gather(idx1, rows1, gsem1)

            drain_gather(rows0, gsem0)
            fire_out(g, rows0, osem0)

            @pl.when(t < half_t - 1)
            def _():
                drain_out(rows0, osem0)
                load_idx(g + 2, idx0)
                fire_gather(idx0, rows0, gsem0)

            drain_gather(rows1, gsem1)
            fire_out(g + 1, rows1, osem1)
            return carry

        lax.fori_loop(0, half_t, body, 0)
        drain_out(rows0, osem0)
        drain_out(rows1, osem1)

    return k(table_pad, idx)


def kernel(atomic_numbers, embedding_table):
    n_outer, rows_i = atomic_numbers.shape
    total = atomic_numbers.size
    idx = atomic_numbers.reshape(total).astype(jnp.int32)
    table_pad = jnp.pad(embedding_table, ((0, 0), (0, EMB)))
    rows = _sc_embedding_gather(table_pad, idx, total)
    return rows.reshape(n_outer, rows_i, 2 * EMB)[:, :, :EMB]


# speedup vs baseline: 2.6736x; 2.6736x over previous
import functools

import jax
import jax.numpy as jnp
from jax import lax
from jax.experimental import pallas as pl
from jax.experimental.pallas import tpu as pltpu
from jax.experimental.pallas import tpu_sc as plsc

EMB = 64
CHUNK = 400
SUBS = ((0, 128), (128, 128), (256, 128), (384, 16))
REPL = 128


@functools.partial(jax.jit, static_argnames=("total",))
def _sc_embedding_gather(table_pad, idx, total):
    info = plsc.get_sparse_core_info()
    num_workers = info.num_cores * info.num_subcores
    per_worker = total // num_workers
    n_chunks = per_worker // CHUNK
    half_t = n_chunks // 2
    mesh = plsc.VectorSubcoreMesh(core_axis_name="c", subcore_axis_name="s")

    @functools.partial(
        pl.kernel,
        mesh=mesh,
        out_type=jax.ShapeDtypeStruct((total, 2 * EMB), jnp.float32),
        scratch_types=[
            pltpu.VMEM((CHUNK,), jnp.int32),
            pltpu.VMEM((CHUNK,), jnp.int32),
            pltpu.VMEM((CHUNK, 2 * EMB), jnp.float32),
            pltpu.VMEM((CHUNK, 2 * EMB), jnp.float32),
            pltpu.SemaphoreType.DMA,
            pltpu.SemaphoreType.DMA,
            pltpu.SemaphoreType.DMA,
            pltpu.SemaphoreType.DMA,
        ],
    )
    def k(pt_hbm, idx_hbm, out_hbm, idx0, idx1, rows0, rows1,
          gsem0, gsem1, osem0, osem1):
        wid = lax.axis_index("s") * info.num_cores + lax.axis_index("c")
        base = wid * per_worker

        def fire_gather(idx_v, rows_v, gsem):
            for off, sz in SUBS:
                pltpu.async_copy(
                    pt_hbm.at[idx_v.at[pl.ds(off, sz)]],
                    rows_v.at[pl.ds(off, sz)],
                    gsem,
                )

        def drain_gather(rows_v, gsem):
            pltpu.make_async_copy(out_hbm.at[pl.ds(0, CHUNK)], rows_v, gsem).wait()

        def drain_out(rows_v, osem):
            pltpu.make_async_copy(rows_v, out_hbm.at[pl.ds(0, CHUNK)], osem).wait()

        def load_idx(g, idx_v):
            pltpu.sync_copy(idx_hbm.at[pl.ds(base + g * CHUNK, CHUNK)], idx_v)

        def fire_out(g, rows_v, osem):
            pltpu.async_copy(rows_v, out_hbm.at[pl.ds(base + g * CHUNK, CHUNK)], osem)

        load_idx(0, idx0)
        fire_gather(idx0, rows0, gsem0)

        def body(t, carry):
            g = 2 * t

            @pl.when(t > 0)
            def _():
                drain_out(rows1, osem1)

            load_idx(g + 1, idx1)
            fire_gather(idx1, rows1, gsem1)

            drain_gather(rows0, gsem0)
            fire_out(g, rows0, osem0)

            @pl.when(t < half_t - 1)
            def _():
                drain_out(rows0, osem0)
                load_idx(g + 2, idx0)
                fire_gather(idx0, rows0, gsem0)

            drain_gather(rows1, gsem1)
            fire_out(g + 1, rows1, osem1)
            return carry

        lax.fori_loop(0, half_t, body, 0)
        drain_out(rows0, osem0)
        drain_out(rows1, osem1)

    return k(table_pad, idx)


def kernel(atomic_numbers, embedding_table):
    n_outer, rows_i = atomic_numbers.shape
    total = atomic_numbers.size
    v = embedding_table.shape[0]
    idx = atomic_numbers.reshape(total).astype(jnp.int32)
    idx = idx + v * (jnp.arange(total, dtype=jnp.int32) % REPL)
    table_pad = jnp.pad(embedding_table, ((0, 0), (0, EMB)))
    table_rep = jnp.tile(table_pad, (REPL, 1))
    rows = _sc_embedding_gather(table_rep, idx, total)
    return rows.reshape(n_outer, rows_i, 2 * EMB)[:, :, :EMB]
